# baseline (speedup 1.0000x reference)
import jax
import jax.numpy as jnp
from jax import lax
from jax.experimental import pallas as pl
from jax.experimental.pallas import tpu as pltpu
from jax.experimental.pallas import tpu_sc as plsc

B = 16384
L = 20
D = 64
NU = 1000001

NC = 2
NS = 16
NW = NC * NS
B_PER_W = B // NW
CH = 32
N_CHUNKS = B_PER_W // CH
IDX_ROWS = CH * L // 128
UG = CH * D // 128


def _sc_body(ing_flat, user_h, recipe_h, ing_table, ut_flat, user_bias,
             recipe_bias, out_h,
             idx0, idx1, uidx0, uidx1, ridx0, ridx1,
             rows0, rows1, uflatidx0, uflatidx1, uval0, uval1,
             ub0, ub1, rb0, rb1,
             out_v, m_v, semg0, semg1, semi0, semi1):
    wid = lax.axis_index("s") * NC + lax.axis_index("c")
    lane = lax.iota(jnp.int32, 16)

    idxb = [idx0, idx1]
    uidxb = [uidx0, uidx1]
    ridxb = [ridx0, ridx1]
    rowsb = [rows0, rows1]
    uflatidxb = [uflatidx0, uflatidx1]
    uvalb = [uval0, uval1]
    ubb = [ub0, ub1]
    rbb = [rb0, rb1]
    semg = [semg0, semg1]
    semi = [semi0, semi1]

    def fire_idx(c, p):
        base = wid * B_PER_W + c * CH
        pltpu.async_copy(ing_flat.at[pl.ds(base * L, CH * L)], idxb[p],
                         semi[p])
        pltpu.async_copy(user_h.at[pl.ds(base, CH)], uidxb[p], semi[p])
        pltpu.async_copy(recipe_h.at[pl.ds(base, CH)], ridxb[p], semi[p])

    def wait_idx(p):
        pltpu.make_async_copy(ing_flat.at[pl.ds(0, CH * L)], idxb[p],
                              semi[p]).wait()
        pltpu.make_async_copy(user_h.at[pl.ds(0, CH)], uidxb[p],
                              semi[p]).wait()
        pltpu.make_async_copy(recipe_h.at[pl.ds(0, CH)], ridxb[p],
                              semi[p]).wait()

    def build_user_idx(p):
        u0 = uidxb[p][pl.ds(0, 16)]
        u1 = uidxb[p][pl.ds(16, 16)]
        for d in range(D):
            uflatidxb[p][pl.ds(d * CH, 16)] = u0 + d * NU
            uflatidxb[p][pl.ds(d * CH + 16, 16)] = u1 + d * NU

    def fire_gathers(p):
        for j in range(IDX_ROWS):
            pltpu.async_copy(
                ing_table.at[idxb[p].at[pl.ds(j * 128, 128)]],
                rowsb[p].at[pl.ds(j * 128, 128)], semg[p])
        for j in range(UG):
            pltpu.async_copy(
                ut_flat.at[uflatidxb[p].at[pl.ds(j * 128, 128)]],
                uvalb[p].at[pl.ds(j * 128, 128)], semg[p])
        pltpu.async_copy(user_bias.at[uidxb[p]], ubb[p], semg[p])
        pltpu.async_copy(recipe_bias.at[ridxb[p]], rbb[p], semg[p])

    def wait_gathers(p):
        for j in range(IDX_ROWS):
            pltpu.make_async_copy(
                ing_table.at[pl.ds(0, 128)],
                rowsb[p].at[pl.ds(j * 128, 128)], semg[p]).wait()
        for j in range(UG):
            pltpu.make_async_copy(
                ut_flat.at[pl.ds(0, 128)],
                uvalb[p].at[pl.ds(j * 128, 128)], semg[p]).wait()
        pltpu.make_async_copy(user_bias.at[pl.ds(0, CH)], ubb[p],
                              semg[p]).wait()
        pltpu.make_async_copy(recipe_bias.at[pl.ds(0, CH)], rbb[p],
                              semg[p]).wait()

    ugidx = [(lax.iota(jnp.int32, 16) + dblk * 16) * CH
             for dblk in range(D // 16)]

    def compute(c, p):
        rows_v = rowsb[p]
        uval_v = uvalb[p]
        for g in range(CH // 16):
            def b_body(jj, carry, g=g):
                b = g * 16 + jj
                r = b * L
                v = jnp.zeros((16,), jnp.float32)
                for dblk in range(D // 16):
                    s = rows_v[r, pl.ds(dblk * 16, 16)]
                    for l in range(1, L):
                        s = s + rows_v[r + l, pl.ds(dblk * 16, 16)]
                    u = plsc.load_gather(uval_v, [ugidx[dblk] + b])
                    v = v + s * u
                m_v[jj] = v
                return carry

            lax.fori_loop(0, 16, b_body, 0)
            score_vec = plsc.load_gather(
                m_v, [lane, jnp.zeros((16,), jnp.int32)])
            for i in range(1, 16):
                score_vec = score_vec + plsc.load_gather(
                    m_v, [lane, jnp.full((16,), i, jnp.int32)])
            score_vec = (score_vec + ubb[p][pl.ds(g * 16, 16)]
                         + rbb[p][pl.ds(g * 16, 16)])
            out_v[pl.ds(c * CH + g * 16, 16)] = score_vec

    fire_idx(0, 0)
    fire_idx(1, 1)
    wait_idx(0)
    build_user_idx(0)
    fire_gathers(0)

    def body(i, _):
        c0 = 2 * i
        wait_gathers(0)
        wait_idx(1)
        build_user_idx(1)
        fire_gathers(1)
        fire_idx(c0 + 2, 0)
        compute(c0, 0)

        wait_gathers(1)
        wait_idx(0)
        build_user_idx(0)
        fire_gathers(0)
        fire_idx(c0 + 3, 1)
        compute(c0 + 1, 1)
        return 0

    lax.fori_loop(0, (N_CHUNKS - 2) // 2, body, 0)

    wait_gathers(0)
    wait_idx(1)
    build_user_idx(1)
    fire_gathers(1)
    compute(N_CHUNKS - 2, 0)
    wait_gathers(1)
    compute(N_CHUNKS - 1, 1)

    pltpu.sync_copy(out_v, out_h.at[pl.ds(wid * B_PER_W, B_PER_W)])


@jax.jit
def _run(ing_flat, user, recipe, ing_table, ut_flat, user_bias,
         recipe_bias):
    mesh = plsc.VectorSubcoreMesh(core_axis_name="c", subcore_axis_name="s")
    return pl.kernel(
        _sc_body,
        out_type=jax.ShapeDtypeStruct((B,), jnp.float32),
        mesh=mesh,
        compiler_params=pltpu.CompilerParams(
            needs_layout_passes=False, use_tc_tiling_on_sc=False),
        scratch_types=[
            pltpu.VMEM((CH * L,), jnp.int32),
            pltpu.VMEM((CH * L,), jnp.int32),
            pltpu.VMEM((CH,), jnp.int32),
            pltpu.VMEM((CH,), jnp.int32),
            pltpu.VMEM((CH,), jnp.int32),
            pltpu.VMEM((CH,), jnp.int32),
            pltpu.VMEM((CH * L, D), jnp.float32),
            pltpu.VMEM((CH * L, D), jnp.float32),
            pltpu.VMEM((CH * D,), jnp.int32),
            pltpu.VMEM((CH * D,), jnp.int32),
            pltpu.VMEM((CH * D,), jnp.float32),
            pltpu.VMEM((CH * D,), jnp.float32),
            pltpu.VMEM((CH,), jnp.float32),
            pltpu.VMEM((CH,), jnp.float32),
            pltpu.VMEM((CH,), jnp.float32),
            pltpu.VMEM((CH,), jnp.float32),
            pltpu.VMEM((B_PER_W,), jnp.float32),
            pltpu.VMEM((16, 16), jnp.float32),
            pltpu.SemaphoreType.DMA,
            pltpu.SemaphoreType.DMA,
            pltpu.SemaphoreType.DMA,
            pltpu.SemaphoreType.DMA,
        ],
    )(ing_flat, user, recipe, ing_table, ut_flat, user_bias, recipe_bias)


def kernel(ingredients, user, recipe, user_table, ing_table, user_bias,
           recipe_bias):
    ing_flat = ingredients.reshape(-1)
    ut_flat = user_table.T.reshape(-1)
    return _run(ing_flat, user, recipe, ing_table, ut_flat,
                user_bias.reshape(-1), recipe_bias.reshape(-1))


# speedup vs baseline: 9.5191x; 9.5191x over previous
import jax
import jax.numpy as jnp
from jax import lax
from jax.experimental import pallas as pl
from jax.experimental.pallas import tpu as pltpu
from jax.experimental.pallas import tpu_sc as plsc

B = 16384
L = 20
D = 64

NC = 2
NS = 16
NW = NC * NS
B_PER_W = B // NW
CH = 32
N_CHUNKS = B_PER_W // CH
IDX_ROWS = CH * L // 128


def _ing_body(ing_flat, user_h, recipe_h, ing_table, user_bias, recipe_bias,
              remb_out, pb_out,
              idx0, idx1, uidx0, uidx1, ridx0, ridx1,
              rows0, rows1, remb0, remb1, ub0, ub1, rb0, rb1,
              semg0, semg1, semi0, semi1):
    wid = lax.axis_index("s") * NC + lax.axis_index("c")

    idxb = [idx0, idx1]
    uidxb = [uidx0, uidx1]
    ridxb = [ridx0, ridx1]
    rowsb = [rows0, rows1]
    rembb = [remb0, remb1]
    ubb = [ub0, ub1]
    rbb = [rb0, rb1]
    semg = [semg0, semg1]
    semi = [semi0, semi1]

    def fire_idx(c, p):
        base = wid * B_PER_W + c * CH
        pltpu.async_copy(ing_flat.at[pl.ds(base * L, CH * L)], idxb[p],
                         semi[p])
        pltpu.async_copy(user_h.at[pl.ds(base, CH)], uidxb[p], semi[p])
        pltpu.async_copy(recipe_h.at[pl.ds(base, CH)], ridxb[p], semi[p])

    def wait_idx(p):
        pltpu.make_async_copy(ing_flat.at[pl.ds(0, CH * L)], idxb[p],
                              semi[p]).wait()
        pltpu.make_async_copy(user_h.at[pl.ds(0, CH)], uidxb[p],
                              semi[p]).wait()
        pltpu.make_async_copy(recipe_h.at[pl.ds(0, CH)], ridxb[p],
                              semi[p]).wait()

    def fire_gathers(p):
        for j in range(IDX_ROWS):
            pltpu.async_copy(
                ing_table.at[idxb[p].at[pl.ds(j * 128, 128)]],
                rowsb[p].at[pl.ds(j * 128, 128)], semg[p])
        pltpu.async_copy(user_bias.at[uidxb[p]], ubb[p], semg[p])
        pltpu.async_copy(recipe_bias.at[ridxb[p]], rbb[p], semg[p])

    def wait_gathers(p):
        for j in range(IDX_ROWS):
            pltpu.make_async_copy(
                ing_table.at[pl.ds(0, 128)],
                rowsb[p].at[pl.ds(j * 128, 128)], semg[p]).wait()
        pltpu.make_async_copy(user_bias.at[pl.ds(0, CH)], ubb[p],
                              semg[p]).wait()
        pltpu.make_async_copy(recipe_bias.at[pl.ds(0, CH)], rbb[p],
                              semg[p]).wait()

    def compute(c, p):
        rows_v = rowsb[p]
        remb_v = rembb[p]
        base = wid * B_PER_W + c * CH

        def b_body(b, carry):
            r = b * L
            for dblk in range(D // 16):
                s = rows_v[r, pl.ds(dblk * 16, 16)]
                for l in range(1, L):
                    s = s + rows_v[r + l, pl.ds(dblk * 16, 16)]
                remb_v[pl.ds(b * D + dblk * 16, 16)] = s
            return carry

        lax.fori_loop(0, CH, b_body, 0)
        pltpu.sync_copy(remb_v, remb_out.at[pl.ds(base * D, CH * D)])
        for g in range(CH // 16):
            pb = (ubb[p][pl.ds(g * 16, 16)] + rbb[p][pl.ds(g * 16, 16)])
            rembb[p][pl.ds(g * 16, 16)] = pb
        pltpu.sync_copy(rembb[p].at[pl.ds(0, CH)],
                        pb_out.at[pl.ds(base, CH)])

    fire_idx(0, 0)
    fire_idx(1, 1)
    wait_idx(0)
    fire_gathers(0)

    def body(i, _):
        c0 = 2 * i
        wait_gathers(0)
        wait_idx(1)
        fire_gathers(1)
        fire_idx(c0 + 2, 0)
        compute(c0, 0)

        wait_gathers(1)
        wait_idx(0)
        fire_gathers(0)
        fire_idx(c0 + 3, 1)
        compute(c0 + 1, 1)
        return 0

    lax.fori_loop(0, (N_CHUNKS - 2) // 2, body, 0)

    wait_gathers(0)
    wait_idx(1)
    fire_gathers(1)
    compute(N_CHUNKS - 2, 0)
    wait_gathers(1)
    compute(N_CHUNKS - 1, 1)


def _user_body(ut, user_h, remb_flat, pb_flat, out_h,
               uidx0, uidx1, low0, low1, remb0, remb1, ugrp0, ugrp1,
               pb0, pb1, out_v, m_v, semg0, semg1, semi0, semi1):
    wid = lax.axis_index("s") * NC + lax.axis_index("c")
    lane = lax.iota(jnp.int32, 16)

    uidxb = [uidx0, uidx1]
    lowb = [low0, low1]
    rembb = [remb0, remb1]
    ugrpb = [ugrp0, ugrp1]
    pbb = [pb0, pb1]
    semg = [semg0, semg1]
    semi = [semi0, semi1]

    def fire_idx(c, p):
        base = wid * B_PER_W + c * CH
        pltpu.async_copy(user_h.at[pl.ds(base, CH)], uidxb[p], semi[p])
        pltpu.async_copy(remb_flat.at[pl.ds(base * D, CH * D)], rembb[p],
                         semi[p])
        pltpu.async_copy(pb_flat.at[pl.ds(base, CH)], pbb[p], semi[p])

    def wait_idx(p):
        pltpu.make_async_copy(user_h.at[pl.ds(0, CH)], uidxb[p],
                              semi[p]).wait()
        pltpu.make_async_copy(remb_flat.at[pl.ds(0, CH * D)], rembb[p],
                              semi[p]).wait()
        pltpu.make_async_copy(pb_flat.at[pl.ds(0, CH)], pbb[p],
                              semi[p]).wait()

    def fire_user(p):
        uvs = [uidxb[p][pl.ds(0, 16)], uidxb[p][pl.ds(16, 16)]]
        lowb[p][pl.ds(0, 16)] = uvs[0] & 7
        lowb[p][pl.ds(16, 16)] = uvs[1] & 7
        for jj in range(CH):
            uid = uvs[jj // 16][jj % 16]
            start = (uid >> 3) * 8
            pltpu.async_copy(ut.at[pl.ds(start, 8)],
                             ugrpb[p].at[pl.ds(jj * 8, 8)], semg[p])

    def wait_user(p):
        for jj in range(CH):
            pltpu.make_async_copy(ut.at[pl.ds(0, 8)],
                                  ugrpb[p].at[pl.ds(jj * 8, 8)],
                                  semg[p]).wait()

    def compute(c, p):
        remb_v = rembb[p]
        ugrp_v = ugrpb[p]
        lows = [lowb[p][pl.ds(0, 16)], lowb[p][pl.ds(16, 16)]]
        for g in range(CH // 16):
            for jj in range(16):
                b = g * 16 + jj
                lo = lows[g][jj]
                v = (remb_v[pl.ds(b * D, 16)]
                     * ugrp_v[b * 8 + lo, pl.ds(0, 16)])
                for dblk in range(1, D // 16):
                    v = v + (remb_v[pl.ds(b * D + dblk * 16, 16)]
                             * ugrp_v[b * 8 + lo, pl.ds(dblk * 16, 16)])
                m_v[pl.ds(jj * 16, 16)] = v
            score_vec = plsc.load_gather(m_v, [lane * 16])
            for i in range(1, 16):
                score_vec = score_vec + plsc.load_gather(
                    m_v, [lane * 16 + i])
            score_vec = score_vec + pbb[p][pl.ds(g * 16, 16)]
            out_v[pl.ds(c * CH + g * 16, 16)] = score_vec

    fire_idx(0, 0)
    fire_idx(1, 1)
    wait_idx(0)
    fire_user(0)

    def body(i, _):
        c0 = 2 * i
        wait_user(0)
        wait_idx(1)
        fire_user(1)
        fire_idx(c0 + 2, 0)
        compute(c0, 0)

        wait_user(1)
        wait_idx(0)
        fire_user(0)
        fire_idx(c0 + 3, 1)
        compute(c0 + 1, 1)
        return 0

    lax.fori_loop(0, (N_CHUNKS - 2) // 2, body, 0)

    wait_user(0)
    wait_idx(1)
    fire_user(1)
    compute(N_CHUNKS - 2, 0)
    wait_user(1)
    compute(N_CHUNKS - 1, 1)

    pltpu.sync_copy(out_v, out_h.at[pl.ds(wid * B_PER_W, B_PER_W)])


@jax.jit
def _run(ing_flat, user, recipe, ing_table, user_table, user_bias,
         recipe_bias):
    mesh = plsc.VectorSubcoreMesh(core_axis_name="c", subcore_axis_name="s")
    remb_flat, pb_flat = pl.kernel(
        _ing_body,
        out_type=(jax.ShapeDtypeStruct((B * D,), jnp.float32),
                  jax.ShapeDtypeStruct((B,), jnp.float32)),
        mesh=mesh,
        compiler_params=pltpu.CompilerParams(
            needs_layout_passes=False, use_tc_tiling_on_sc=False),
        scratch_types=[
            pltpu.VMEM((CH * L,), jnp.int32),
            pltpu.VMEM((CH * L,), jnp.int32),
            pltpu.VMEM((CH,), jnp.int32),
            pltpu.VMEM((CH,), jnp.int32),
            pltpu.VMEM((CH,), jnp.int32),
            pltpu.VMEM((CH,), jnp.int32),
            pltpu.VMEM((CH * L, D), jnp.float32),
            pltpu.VMEM((CH * L, D), jnp.float32),
            pltpu.VMEM((CH * D,), jnp.float32),
            pltpu.VMEM((CH * D,), jnp.float32),
            pltpu.VMEM((CH,), jnp.float32),
            pltpu.VMEM((CH,), jnp.float32),
            pltpu.VMEM((CH,), jnp.float32),
            pltpu.VMEM((CH,), jnp.float32),
            pltpu.SemaphoreType.DMA,
            pltpu.SemaphoreType.DMA,
            pltpu.SemaphoreType.DMA,
            pltpu.SemaphoreType.DMA,
        ],
    )(ing_flat, user, recipe, ing_table, user_bias, recipe_bias)

    return pl.kernel(
        _user_body,
        out_type=jax.ShapeDtypeStruct((B,), jnp.float32),
        mesh=mesh,
        compiler_params=pltpu.CompilerParams(
            needs_layout_passes=False, use_tc_tiling_on_sc=True),
        scratch_types=[
            pltpu.VMEM((CH,), jnp.int32),
            pltpu.VMEM((CH,), jnp.int32),
            pltpu.VMEM((CH,), jnp.int32),
            pltpu.VMEM((CH,), jnp.int32),
            pltpu.VMEM((CH * D,), jnp.float32),
            pltpu.VMEM((CH * D,), jnp.float32),
            pltpu.VMEM((CH * 8, D), jnp.float32),
            pltpu.VMEM((CH * 8, D), jnp.float32),
            pltpu.VMEM((CH,), jnp.float32),
            pltpu.VMEM((CH,), jnp.float32),
            pltpu.VMEM((B_PER_W,), jnp.float32),
            pltpu.VMEM((256,), jnp.float32),
            pltpu.SemaphoreType.DMA,
            pltpu.SemaphoreType.DMA,
            pltpu.SemaphoreType.DMA,
            pltpu.SemaphoreType.DMA,
        ],
    )(user_table, user, remb_flat, pb_flat)


def kernel(ingredients, user, recipe, user_table, ing_table, user_bias,
           recipe_bias):
    ing_flat = ingredients.reshape(-1)
    return _run(ing_flat, user, recipe, ing_table, user_table,
                user_bias.reshape(-1), recipe_bias.reshape(-1))


# speedup vs baseline: 9.5458x; 1.0028x over previous
import jax
import jax.numpy as jnp
from jax import lax
from jax.experimental import pallas as pl
from jax.experimental.pallas import tpu as pltpu
from jax.experimental.pallas import tpu_sc as plsc

B = 16384
L = 20
D = 64

NC = 2
NS = 16
NW = NC * NS
B_PER_W = B // NW
CH = 32
N_CHUNKS = B_PER_W // CH
IDX_ROWS = CH * L // 128


def _ing_body(ing_flat, user_h, recipe_h, ing_table, user_bias, recipe_bias,
              remb_out, pb_out,
              idx0, idx1, uidx0, uidx1, ridx0, ridx1,
              rows0, rows1, remb0, remb1, pb0, pb1, ub0, ub1, rb0, rb1,
              semg0, semg1, semi0, semi1, semo0, semo1):
    wid = lax.axis_index("s") * NC + lax.axis_index("c")

    idxb = [idx0, idx1]
    uidxb = [uidx0, uidx1]
    ridxb = [ridx0, ridx1]
    rowsb = [rows0, rows1]
    rembb = [remb0, remb1]
    pbb = [pb0, pb1]
    ubb = [ub0, ub1]
    rbb = [rb0, rb1]
    semg = [semg0, semg1]
    semi = [semi0, semi1]
    semo = [semo0, semo1]

    def fire_idx(c, p):
        base = wid * B_PER_W + c * CH
        pltpu.async_copy(ing_flat.at[pl.ds(base * L, CH * L)], idxb[p],
                         semi[p])
        pltpu.async_copy(user_h.at[pl.ds(base, CH)], uidxb[p], semi[p])
        pltpu.async_copy(recipe_h.at[pl.ds(base, CH)], ridxb[p], semi[p])

    def wait_idx(p):
        pltpu.make_async_copy(ing_flat.at[pl.ds(0, CH * L)], idxb[p],
                              semi[p]).wait()
        pltpu.make_async_copy(user_h.at[pl.ds(0, CH)], uidxb[p],
                              semi[p]).wait()
        pltpu.make_async_copy(recipe_h.at[pl.ds(0, CH)], ridxb[p],
                              semi[p]).wait()

    def fire_gathers(p):
        for j in range(IDX_ROWS):
            pltpu.async_copy(
                ing_table.at[idxb[p].at[pl.ds(j * 128, 128)]],
                rowsb[p].at[pl.ds(j * 128, 128)], semg[p])
        pltpu.async_copy(user_bias.at[uidxb[p]], ubb[p], semg[p])
        pltpu.async_copy(recipe_bias.at[ridxb[p]], rbb[p], semg[p])

    def wait_gathers(p):
        for j in range(IDX_ROWS):
            pltpu.make_async_copy(
                ing_table.at[pl.ds(0, 128)],
                rowsb[p].at[pl.ds(j * 128, 128)], semg[p]).wait()
        pltpu.make_async_copy(user_bias.at[pl.ds(0, CH)], ubb[p],
                              semg[p]).wait()
        pltpu.make_async_copy(recipe_bias.at[pl.ds(0, CH)], rbb[p],
                              semg[p]).wait()

    def fire_out(c, p):
        base = wid * B_PER_W + c * CH
        pltpu.async_copy(rembb[p], remb_out.at[pl.ds(base * D, CH * D)],
                         semo[p])
        pltpu.async_copy(pbb[p], pb_out.at[pl.ds(base, CH)], semo[p])

    def drain_out(p):
        pltpu.make_async_copy(rembb[p], remb_out.at[pl.ds(0, CH * D)],
                              semo[p]).wait()
        pltpu.make_async_copy(pbb[p], pb_out.at[pl.ds(0, CH)],
                              semo[p]).wait()

    def compute(c, p):
        rows_v = rowsb[p]
        remb_v = rembb[p]
        drain_out(p)

        def b_body(b, carry):
            r = b * L
            for dblk in range(D // 16):
                s = rows_v[r, pl.ds(dblk * 16, 16)]
                for l in range(1, L):
                    s = s + rows_v[r + l, pl.ds(dblk * 16, 16)]
                remb_v[pl.ds(b * D + dblk * 16, 16)] = s
            return carry

        lax.fori_loop(0, CH, b_body, 0, unroll=2)
        for g in range(CH // 16):
            pbb[p][pl.ds(g * 16, 16)] = (ubb[p][pl.ds(g * 16, 16)]
                                         + rbb[p][pl.ds(g * 16, 16)])
        fire_out(c, p)

    fire_idx(0, 0)
    fire_idx(1, 1)
    fire_out(0, 0)
    fire_out(1, 1)
    wait_idx(0)
    fire_gathers(0)

    def body(i, _):
        c0 = 2 * i
        wait_gathers(0)
        wait_idx(1)
        fire_gathers(1)
        fire_idx(c0 + 2, 0)
        compute(c0, 0)

        wait_gathers(1)
        wait_idx(0)
        fire_gathers(0)
        fire_idx(c0 + 3, 1)
        compute(c0 + 1, 1)
        return 0

    lax.fori_loop(0, (N_CHUNKS - 2) // 2, body, 0)

    wait_gathers(0)
    wait_idx(1)
    fire_gathers(1)
    compute(N_CHUNKS - 2, 0)
    wait_gathers(1)
    compute(N_CHUNKS - 1, 1)
    drain_out(0)
    drain_out(1)


def _user_body(ut, user_h, remb_flat, pb_flat, out_h,
               uidx0, uidx1, low0, low1, remb0, remb1, ugrp0, ugrp1,
               pb0, pb1, out_v, m_v, semg0, semg1, semi0, semi1):
    wid = lax.axis_index("s") * NC + lax.axis_index("c")
    lane = lax.iota(jnp.int32, 16)

    uidxb = [uidx0, uidx1]
    lowb = [low0, low1]
    rembb = [remb0, remb1]
    ugrpb = [ugrp0, ugrp1]
    pbb = [pb0, pb1]
    semg = [semg0, semg1]
    semi = [semi0, semi1]

    def fire_idx(c, p):
        base = wid * B_PER_W + c * CH
        pltpu.async_copy(user_h.at[pl.ds(base, CH)], uidxb[p], semi[p])
        pltpu.async_copy(remb_flat.at[pl.ds(base * D, CH * D)], rembb[p],
                         semi[p])
        pltpu.async_copy(pb_flat.at[pl.ds(base, CH)], pbb[p], semi[p])

    def wait_idx(p):
        pltpu.make_async_copy(user_h.at[pl.ds(0, CH)], uidxb[p],
                              semi[p]).wait()
        pltpu.make_async_copy(remb_flat.at[pl.ds(0, CH * D)], rembb[p],
                              semi[p]).wait()
        pltpu.make_async_copy(pb_flat.at[pl.ds(0, CH)], pbb[p],
                              semi[p]).wait()

    def fire_user(p):
        uvs = [uidxb[p][pl.ds(0, 16)], uidxb[p][pl.ds(16, 16)]]
        lowb[p][pl.ds(0, 16)] = uvs[0] & 7
        lowb[p][pl.ds(16, 16)] = uvs[1] & 7
        for jj in range(CH):
            uid = uvs[jj // 16][jj % 16]
            start = (uid >> 3) * 8
            pltpu.async_copy(ut.at[pl.ds(start, 8)],
                             ugrpb[p].at[pl.ds(jj * 8, 8)], semg[p])

    def wait_user(p):
        for jj in range(CH):
            pltpu.make_async_copy(ut.at[pl.ds(0, 8)],
                                  ugrpb[p].at[pl.ds(jj * 8, 8)],
                                  semg[p]).wait()

    def compute(c, p):
        remb_v = rembb[p]
        ugrp_v = ugrpb[p]
        lows = [lowb[p][pl.ds(0, 16)], lowb[p][pl.ds(16, 16)]]
        for g in range(CH // 16):
            for jj in range(16):
                b = g * 16 + jj
                lo = lows[g][jj]
                v = (remb_v[pl.ds(b * D, 16)]
                     * ugrp_v[b * 8 + lo, pl.ds(0, 16)])
                for dblk in range(1, D // 16):
                    v = v + (remb_v[pl.ds(b * D + dblk * 16, 16)]
                             * ugrp_v[b * 8 + lo, pl.ds(dblk * 16, 16)])
                m_v[pl.ds(jj * 16, 16)] = v
            score_vec = plsc.load_gather(m_v, [lane * 16])
            for i in range(1, 16):
                score_vec = score_vec + plsc.load_gather(
                    m_v, [lane * 16 + i])
            score_vec = score_vec + pbb[p][pl.ds(g * 16, 16)]
            out_v[pl.ds(c * CH + g * 16, 16)] = score_vec

    fire_idx(0, 0)
    fire_idx(1, 1)
    wait_idx(0)
    fire_user(0)

    def body(i, _):
        c0 = 2 * i
        wait_user(0)
        wait_idx(1)
        fire_user(1)
        fire_idx(c0 + 2, 0)
        compute(c0, 0)

        wait_user(1)
        wait_idx(0)
        fire_user(0)
        fire_idx(c0 + 3, 1)
        compute(c0 + 1, 1)
        return 0

    lax.fori_loop(0, (N_CHUNKS - 2) // 2, body, 0)

    wait_user(0)
    wait_idx(1)
    fire_user(1)
    compute(N_CHUNKS - 2, 0)
    wait_user(1)
    compute(N_CHUNKS - 1, 1)

    pltpu.sync_copy(out_v, out_h.at[pl.ds(wid * B_PER_W, B_PER_W)])


@jax.jit
def _run(ing_flat, user, recipe, ing_table, user_table, user_bias,
         recipe_bias):
    mesh = plsc.VectorSubcoreMesh(core_axis_name="c", subcore_axis_name="s")
    remb_flat, pb_flat = pl.kernel(
        _ing_body,
        out_type=(jax.ShapeDtypeStruct((B * D,), jnp.float32),
                  jax.ShapeDtypeStruct((B,), jnp.float32)),
        mesh=mesh,
        compiler_params=pltpu.CompilerParams(
            needs_layout_passes=False, use_tc_tiling_on_sc=False),
        scratch_types=[
            pltpu.VMEM((CH * L,), jnp.int32),
            pltpu.VMEM((CH * L,), jnp.int32),
            pltpu.VMEM((CH,), jnp.int32),
            pltpu.VMEM((CH,), jnp.int32),
            pltpu.VMEM((CH,), jnp.int32),
            pltpu.VMEM((CH,), jnp.int32),
            pltpu.VMEM((CH * L, D), jnp.float32),
            pltpu.VMEM((CH * L, D), jnp.float32),
            pltpu.VMEM((CH * D,), jnp.float32),
            pltpu.VMEM((CH * D,), jnp.float32),
            pltpu.VMEM((CH,), jnp.float32),
            pltpu.VMEM((CH,), jnp.float32),
            pltpu.VMEM((CH,), jnp.float32),
            pltpu.VMEM((CH,), jnp.float32),
            pltpu.VMEM((CH,), jnp.float32),
            pltpu.VMEM((CH,), jnp.float32),
            pltpu.SemaphoreType.DMA,
            pltpu.SemaphoreType.DMA,
            pltpu.SemaphoreType.DMA,
            pltpu.SemaphoreType.DMA,
            pltpu.SemaphoreType.DMA,
            pltpu.SemaphoreType.DMA,
        ],
    )(ing_flat, user, recipe, ing_table, user_bias, recipe_bias)

    return pl.kernel(
        _user_body,
        out_type=jax.ShapeDtypeStruct((B,), jnp.float32),
        mesh=mesh,
        compiler_params=pltpu.CompilerParams(
            needs_layout_passes=False, use_tc_tiling_on_sc=True),
        scratch_types=[
            pltpu.VMEM((CH,), jnp.int32),
            pltpu.VMEM((CH,), jnp.int32),
            pltpu.VMEM((CH,), jnp.int32),
            pltpu.VMEM((CH,), jnp.int32),
            pltpu.VMEM((CH * D,), jnp.float32),
            pltpu.VMEM((CH * D,), jnp.float32),
            pltpu.VMEM((CH * 8, D), jnp.float32),
            pltpu.VMEM((CH * 8, D), jnp.float32),
            pltpu.VMEM((CH,), jnp.float32),
            pltpu.VMEM((CH,), jnp.float32),
            pltpu.VMEM((B_PER_W,), jnp.float32),
            pltpu.VMEM((256,), jnp.float32),
            pltpu.SemaphoreType.DMA,
            pltpu.SemaphoreType.DMA,
            pltpu.SemaphoreType.DMA,
            pltpu.SemaphoreType.DMA,
        ],
    )(user_table, user, remb_flat, pb_flat)


def kernel(ingredients, user, recipe, user_table, ing_table, user_bias,
           recipe_bias):
    ing_flat = ingredients.reshape(-1)
    return _run(ing_flat, user, recipe, ing_table, user_table,
                user_bias.reshape(-1), recipe_bias.reshape(-1))


# speedup vs baseline: 10.9659x; 1.1488x over previous
import jax
import jax.numpy as jnp
from jax import lax
from jax.experimental import pallas as pl
from jax.experimental.pallas import tpu as pltpu
from jax.experimental.pallas import tpu_sc as plsc

B = 16384
L = 20
D = 64

NC = 2
NS = 16
NW = NC * NS
B_PER_W = B // NW
CH = 32
N_CHUNKS = B_PER_W // CH
IDX_ROWS = CH * L // 128


def _ing_body(ing_flat, user_h, recipe_h, ing_table, user_bias, recipe_bias,
              remb_out, pb_out,
              idx0, idx1, uidx0, uidx1, ridx0, ridx1,
              rows0, rows1, remb0, remb1, ub0, ub1, rb0, rb1,
              semg0, semg1, semi0, semi1):
    wid = lax.axis_index("s") * NC + lax.axis_index("c")

    idxb = [idx0, idx1]
    uidxb = [uidx0, uidx1]
    ridxb = [ridx0, ridx1]
    rowsb = [rows0, rows1]
    rembb = [remb0, remb1]
    ubb = [ub0, ub1]
    rbb = [rb0, rb1]
    semg = [semg0, semg1]
    semi = [semi0, semi1]

    def fire_idx(c, p):
        base = wid * B_PER_W + c * CH
        pltpu.async_copy(ing_flat.at[pl.ds(base * L, CH * L)], idxb[p],
                         semi[p])
        pltpu.async_copy(user_h.at[pl.ds(base, CH)], uidxb[p], semi[p])
        pltpu.async_copy(recipe_h.at[pl.ds(base, CH)], ridxb[p], semi[p])

    def wait_idx(p):
        pltpu.make_async_copy(ing_flat.at[pl.ds(0, CH * L)], idxb[p],
                              semi[p]).wait()
        pltpu.make_async_copy(user_h.at[pl.ds(0, CH)], uidxb[p],
                              semi[p]).wait()
        pltpu.make_async_copy(recipe_h.at[pl.ds(0, CH)], ridxb[p],
                              semi[p]).wait()

    def fire_gathers(p):
        for j in range(IDX_ROWS):
            pltpu.async_copy(
                ing_table.at[idxb[p].at[pl.ds(j * 128, 128)]],
                rowsb[p].at[pl.ds(j * 128, 128)], semg[p])
        pltpu.async_copy(user_bias.at[uidxb[p]], ubb[p], semg[p])
        pltpu.async_copy(recipe_bias.at[ridxb[p]], rbb[p], semg[p])

    def wait_gathers(p):
        for j in range(IDX_ROWS):
            pltpu.make_async_copy(
                ing_table.at[pl.ds(0, 128)],
                rowsb[p].at[pl.ds(j * 128, 128)], semg[p]).wait()
        pltpu.make_async_copy(user_bias.at[pl.ds(0, CH)], ubb[p],
                              semg[p]).wait()
        pltpu.make_async_copy(recipe_bias.at[pl.ds(0, CH)], rbb[p],
                              semg[p]).wait()

    def compute(c, p):
        rows_v = rowsb[p]
        remb_v = rembb[p]
        base = wid * B_PER_W + c * CH

        def b_body(b, carry):
            r = b * L
            for dblk in range(D // 16):
                s = rows_v[r, pl.ds(dblk * 16, 16)]
                for l in range(1, L):
                    s = s + rows_v[r + l, pl.ds(dblk * 16, 16)]
                remb_v[pl.ds(b * D + dblk * 16, 16)] = s
            return carry

        lax.fori_loop(0, CH, b_body, 0)
        pltpu.sync_copy(remb_v, remb_out.at[pl.ds(base * D, CH * D)])
        for g in range(CH // 16):
            pb = (ubb[p][pl.ds(g * 16, 16)] + rbb[p][pl.ds(g * 16, 16)])
            rembb[p][pl.ds(g * 16, 16)] = pb
        pltpu.sync_copy(rembb[p].at[pl.ds(0, CH)],
                        pb_out.at[pl.ds(base, CH)])

    fire_idx(0, 0)
    fire_idx(1, 1)
    wait_idx(0)
    fire_gathers(0)

    def body(i, _):
        c0 = 2 * i
        wait_gathers(0)
        wait_idx(1)
        fire_gathers(1)
        fire_idx(c0 + 2, 0)
        compute(c0, 0)

        wait_gathers(1)
        wait_idx(0)
        fire_gathers(0)
        fire_idx(c0 + 3, 1)
        compute(c0 + 1, 1)
        return 0

    lax.fori_loop(0, (N_CHUNKS - 2) // 2, body, 0)

    wait_gathers(0)
    wait_idx(1)
    fire_gathers(1)
    compute(N_CHUNKS - 2, 0)
    wait_gathers(1)
    compute(N_CHUNKS - 1, 1)


SCH = 16
NSCH = B_PER_W // SCH
Q = 4


def _user_body(ut_t, user_h, remb_flat, pb_flat, out_h,
               uidx0, uidx1, remb0, remb1, band0, band1,
               pb0, pb1, out_v, m_v, semg0, semg1, semi0, semi1):
    wid = lax.axis_index("s") * NC + lax.axis_index("c")
    lane = lax.iota(jnp.int32, 16)

    uidxb = [uidx0, uidx1]
    rembb = [remb0, remb1]
    bandb = [band0, band1]
    pbb = [pb0, pb1]
    semg = [semg0, semg1]
    semi = [semi0, semi1]

    def stage(s, p):
        base = wid * B_PER_W + s * SCH
        pltpu.async_copy(user_h.at[pl.ds(base, SCH)], uidxb[p], semi[p])
        pltpu.async_copy(remb_flat.at[pl.ds(base * D, SCH * D)], rembb[p],
                         semi[p])
        pltpu.async_copy(pb_flat.at[pl.ds(base, SCH)], pbb[p], semi[p])

    def wait_stage(p):
        pltpu.make_async_copy(user_h.at[pl.ds(0, SCH)], uidxb[p],
                              semi[p]).wait()
        pltpu.make_async_copy(remb_flat.at[pl.ds(0, SCH * D)], rembb[p],
                              semi[p]).wait()
        pltpu.make_async_copy(pb_flat.at[pl.ds(0, SCH)], pbb[p],
                              semi[p]).wait()

    def fire_bands(uids, q, pq):
        for k in range(Q):
            col = (uids[q * Q + k] >> 7) * 128
            pltpu.async_copy(
                ut_t.at[pl.ds(0, D), pl.ds(col, 128)],
                bandb[pq].at[pl.ds(k * D, D)], semg[pq])

    def wait_bands(pq):
        for k in range(Q):
            pltpu.make_async_copy(
                ut_t.at[pl.ds(0, D), pl.ds(0, 128)],
                bandb[pq].at[pl.ds(k * D, D)], semg[pq]).wait()

    def compute_quarter(uids, remb_v, q, pq):
        band_v = bandb[pq]
        for k in range(Q):
            b = q * Q + k
            lo = uids[b] & 127
            lovec = jnp.full((16,), lo, jnp.int32)
            v = jnp.zeros((16,), jnp.float32)
            for dblk in range(D // 16):
                u = plsc.load_gather(
                    band_v, [k * D + dblk * 16 + lane, lovec])
                v = v + remb_v[pl.ds(b * D + dblk * 16, 16)] * u
            m_v[pl.ds(b * 16, 16)] = v

    def flush(s, p):
        score_vec = plsc.load_gather(m_v, [lane * 16])
        for i in range(1, 16):
            score_vec = score_vec + plsc.load_gather(m_v, [lane * 16 + i])
        score_vec = score_vec + pbb[p][pl.ds(0, 16)]
        out_v[pl.ds(s * SCH, 16)] = score_vec

    def superchunk(s, p, do_stage_next):
        wait_stage(p)
        uvec = uidxb[p][pl.ds(0, 16)]
        uids = [uvec[j] for j in range(SCH)]
        remb_v = rembb[p]
        if do_stage_next:
            stage(s + 1, 1 - p)
        fire_bands(uids, 0, 0)
        fire_bands(uids, 1, 1)
        wait_bands(0)
        compute_quarter(uids, remb_v, 0, 0)
        fire_bands(uids, 2, 0)
        wait_bands(1)
        compute_quarter(uids, remb_v, 1, 1)
        fire_bands(uids, 3, 1)
        wait_bands(0)
        compute_quarter(uids, remb_v, 2, 0)
        wait_bands(1)
        compute_quarter(uids, remb_v, 3, 1)
        flush(s, p)

    stage(0, 0)

    def body(i, _):
        s0 = 2 * i
        superchunk(s0, 0, True)
        superchunk(s0 + 1, 1, True)
        return 0

    lax.fori_loop(0, (NSCH - 2) // 2, body, 0)
    superchunk(NSCH - 2, 0, True)
    superchunk(NSCH - 1, 1, False)

    pltpu.sync_copy(out_v, out_h.at[pl.ds(wid * B_PER_W, B_PER_W)])


@jax.jit
def _run(ing_flat, user, recipe, ing_table, user_table, user_bias,
         recipe_bias):
    mesh = plsc.VectorSubcoreMesh(core_axis_name="c", subcore_axis_name="s")
    remb_flat, pb_flat = pl.kernel(
        _ing_body,
        out_type=(jax.ShapeDtypeStruct((B * D,), jnp.float32),
                  jax.ShapeDtypeStruct((B,), jnp.float32)),
        mesh=mesh,
        compiler_params=pltpu.CompilerParams(
            needs_layout_passes=False, use_tc_tiling_on_sc=False),
        scratch_types=[
            pltpu.VMEM((CH * L,), jnp.int32),
            pltpu.VMEM((CH * L,), jnp.int32),
            pltpu.VMEM((CH,), jnp.int32),
            pltpu.VMEM((CH,), jnp.int32),
            pltpu.VMEM((CH,), jnp.int32),
            pltpu.VMEM((CH,), jnp.int32),
            pltpu.VMEM((CH * L, D), jnp.float32),
            pltpu.VMEM((CH * L, D), jnp.float32),
            pltpu.VMEM((CH * D,), jnp.float32),
            pltpu.VMEM((CH * D,), jnp.float32),
            pltpu.VMEM((CH,), jnp.float32),
            pltpu.VMEM((CH,), jnp.float32),
            pltpu.VMEM((CH,), jnp.float32),
            pltpu.VMEM((CH,), jnp.float32),
            pltpu.SemaphoreType.DMA,
            pltpu.SemaphoreType.DMA,
            pltpu.SemaphoreType.DMA,
            pltpu.SemaphoreType.DMA,
        ],
    )(ing_flat, user, recipe, ing_table, user_bias, recipe_bias)

    return pl.kernel(
        _user_body,
        out_type=jax.ShapeDtypeStruct((B,), jnp.float32),
        mesh=mesh,
        compiler_params=pltpu.CompilerParams(
            needs_layout_passes=False, use_tc_tiling_on_sc=True),
        scratch_types=[
            pltpu.VMEM((SCH,), jnp.int32),
            pltpu.VMEM((SCH,), jnp.int32),
            pltpu.VMEM((SCH * D,), jnp.float32),
            pltpu.VMEM((SCH * D,), jnp.float32),
            pltpu.VMEM((Q * D, 128), jnp.float32),
            pltpu.VMEM((Q * D, 128), jnp.float32),
            pltpu.VMEM((SCH,), jnp.float32),
            pltpu.VMEM((SCH,), jnp.float32),
            pltpu.VMEM((B_PER_W,), jnp.float32),
            pltpu.VMEM((256,), jnp.float32),
            pltpu.SemaphoreType.DMA,
            pltpu.SemaphoreType.DMA,
            pltpu.SemaphoreType.DMA,
            pltpu.SemaphoreType.DMA,
        ],
    )(user_table, user, remb_flat, pb_flat)


def kernel(ingredients, user, recipe, user_table, ing_table, user_bias,
           recipe_bias):
    ing_flat = ingredients.reshape(-1)
    return _run(ing_flat, user, recipe, ing_table, user_table.T,
                user_bias.reshape(-1), recipe_bias.reshape(-1))


# speedup vs baseline: 11.0051x; 1.0036x over previous
import jax
import jax.numpy as jnp
from jax import lax
from jax.experimental import pallas as pl
from jax.experimental.pallas import tpu as pltpu
from jax.experimental.pallas import tpu_sc as plsc

B = 16384
L = 20
D = 64

NC = 2
NS = 16
NW = NC * NS
B_PER_W = B // NW
CH = 32
N_CHUNKS = B_PER_W // CH
IDX_ROWS = CH * L // 128


def _ing_body(ing_flat, user_h, recipe_h, ing_table, user_bias, recipe_bias,
              remb_out, pb_out,
              idx0, idx1, uidx0, uidx1, ridx0, ridx1,
              rows0, rows1, remb0, remb1, ub0, ub1, rb0, rb1,
              semg0, semg1, semi0, semi1):
    wid = lax.axis_index("s") * NC + lax.axis_index("c")

    idxb = [idx0, idx1]
    uidxb = [uidx0, uidx1]
    ridxb = [ridx0, ridx1]
    rowsb = [rows0, rows1]
    rembb = [remb0, remb1]
    ubb = [ub0, ub1]
    rbb = [rb0, rb1]
    semg = [semg0, semg1]
    semi = [semi0, semi1]

    def fire_idx(c, p):
        base = wid * B_PER_W + c * CH
        pltpu.async_copy(ing_flat.at[pl.ds(base * L, CH * L)], idxb[p],
                         semi[p])
        pltpu.async_copy(user_h.at[pl.ds(base, CH)], uidxb[p], semi[p])
        pltpu.async_copy(recipe_h.at[pl.ds(base, CH)], ridxb[p], semi[p])

    def wait_idx(p):
        pltpu.make_async_copy(ing_flat.at[pl.ds(0, CH * L)], idxb[p],
                              semi[p]).wait()
        pltpu.make_async_copy(user_h.at[pl.ds(0, CH)], uidxb[p],
                              semi[p]).wait()
        pltpu.make_async_copy(recipe_h.at[pl.ds(0, CH)], ridxb[p],
                              semi[p]).wait()

    def fire_gathers(p):
        for j in range(IDX_ROWS):
            pltpu.async_copy(
                ing_table.at[idxb[p].at[pl.ds(j * 128, 128)]],
                rowsb[p].at[pl.ds(j * 128, 128)], semg[p])
        pltpu.async_copy(user_bias.at[uidxb[p]], ubb[p], semg[p])
        pltpu.async_copy(recipe_bias.at[ridxb[p]], rbb[p], semg[p])

    def wait_gathers(p):
        for j in range(IDX_ROWS):
            pltpu.make_async_copy(
                ing_table.at[pl.ds(0, 128)],
                rowsb[p].at[pl.ds(j * 128, 128)], semg[p]).wait()
        pltpu.make_async_copy(user_bias.at[pl.ds(0, CH)], ubb[p],
                              semg[p]).wait()
        pltpu.make_async_copy(recipe_bias.at[pl.ds(0, CH)], rbb[p],
                              semg[p]).wait()

    def compute(c, p):
        rows_v = rowsb[p]
        remb_v = rembb[p]
        base = wid * B_PER_W + c * CH

        def b_body(b, carry):
            r = b * L
            for dblk in range(D // 16):
                s = rows_v[r, pl.ds(dblk * 16, 16)]
                for l in range(1, L):
                    s = s + rows_v[r + l, pl.ds(dblk * 16, 16)]
                remb_v[pl.ds(b * D + dblk * 16, 16)] = s
            return carry

        lax.fori_loop(0, CH, b_body, 0)
        pltpu.sync_copy(remb_v, remb_out.at[pl.ds(base * D, CH * D)])
        for g in range(CH // 16):
            pb = (ubb[p][pl.ds(g * 16, 16)] + rbb[p][pl.ds(g * 16, 16)])
            rembb[p][pl.ds(g * 16, 16)] = pb
        pltpu.sync_copy(rembb[p].at[pl.ds(0, CH)],
                        pb_out.at[pl.ds(base, CH)])

    fire_idx(0, 0)
    fire_idx(1, 1)
    wait_idx(0)
    fire_gathers(0)

    def body(i, _):
        c0 = 2 * i
        wait_gathers(0)
        wait_idx(1)
        fire_gathers(1)
        fire_idx(c0 + 2, 0)
        compute(c0, 0)

        wait_gathers(1)
        wait_idx(0)
        fire_gathers(0)
        fire_idx(c0 + 3, 1)
        compute(c0 + 1, 1)
        return 0

    lax.fori_loop(0, (N_CHUNKS - 2) // 2, body, 0)

    wait_gathers(0)
    wait_idx(1)
    fire_gathers(1)
    compute(N_CHUNKS - 2, 0)
    wait_gathers(1)
    compute(N_CHUNKS - 1, 1)


SCH = 32
NSCH = B_PER_W // SCH
Q = 4
NQ = SCH // Q


def _user_body(ut_t, user_h, remb_flat, pb_flat, out_h,
               uidx0, uidx1, remb0, remb1, band0, band1,
               pb0, pb1, out_v, m_v, semg0, semg1, semi0, semi1):
    wid = lax.axis_index("s") * NC + lax.axis_index("c")
    lane = lax.iota(jnp.int32, 16)

    uidxb = [uidx0, uidx1]
    rembb = [remb0, remb1]
    bandb = [band0, band1]
    pbb = [pb0, pb1]
    semg = [semg0, semg1]
    semi = [semi0, semi1]

    def stage(s, p):
        base = wid * B_PER_W + s * SCH
        pltpu.async_copy(user_h.at[pl.ds(base, SCH)], uidxb[p], semi[p])
        pltpu.async_copy(remb_flat.at[pl.ds(base * D, SCH * D)], rembb[p],
                         semi[p])
        pltpu.async_copy(pb_flat.at[pl.ds(base, SCH)], pbb[p], semi[p])

    def wait_stage(p):
        pltpu.make_async_copy(user_h.at[pl.ds(0, SCH)], uidxb[p],
                              semi[p]).wait()
        pltpu.make_async_copy(remb_flat.at[pl.ds(0, SCH * D)], rembb[p],
                              semi[p]).wait()
        pltpu.make_async_copy(pb_flat.at[pl.ds(0, SCH)], pbb[p],
                              semi[p]).wait()

    def fire_bands(uids, q, pq):
        for k in range(Q):
            col = (uids[q * Q + k] >> 7) * 128
            pltpu.async_copy(
                ut_t.at[pl.ds(0, D), pl.ds(col, 128)],
                bandb[pq].at[pl.ds(k * D, D)], semg[pq])

    def wait_bands(pq):
        for k in range(Q):
            pltpu.make_async_copy(
                ut_t.at[pl.ds(0, D), pl.ds(0, 128)],
                bandb[pq].at[pl.ds(k * D, D)], semg[pq]).wait()

    def compute_quarter(uids, remb_v, q, pq):
        band_v = bandb[pq]
        for k in range(Q):
            b = q * Q + k
            lo = uids[b] & 127
            lovec = jnp.full((16,), lo, jnp.int32)
            v = jnp.zeros((16,), jnp.float32)
            for dblk in range(D // 16):
                u = plsc.load_gather(
                    band_v, [k * D + dblk * 16 + lane, lovec])
                v = v + remb_v[pl.ds(b * D + dblk * 16, 16)] * u
            m_v[pl.ds(b * 16, 16)] = v

    def flush(s, p):
        for g in range(SCH // 16):
            score_vec = plsc.load_gather(m_v, [g * 256 + lane * 16])
            for i in range(1, 16):
                score_vec = score_vec + plsc.load_gather(
                    m_v, [g * 256 + lane * 16 + i])
            score_vec = score_vec + pbb[p][pl.ds(g * 16, 16)]
            out_v[pl.ds(s * SCH + g * 16, 16)] = score_vec

    def superchunk(s, p, do_stage_next):
        wait_stage(p)
        uvecs = [uidxb[p][pl.ds(0, 16)], uidxb[p][pl.ds(16, 16)]]
        uids = [uvecs[j // 16][j % 16] for j in range(SCH)]
        remb_v = rembb[p]
        if do_stage_next:
            stage(s + 1, 1 - p)
        fire_bands(uids, 0, 0)
        fire_bands(uids, 1, 1)
        for q in range(NQ):
            pq = q % 2
            wait_bands(pq)
            compute_quarter(uids, remb_v, q, pq)
            if q + 2 < NQ:
                fire_bands(uids, q + 2, pq)
        flush(s, p)

    stage(0, 0)

    def body(i, _):
        s0 = 2 * i
        superchunk(s0, 0, True)
        superchunk(s0 + 1, 1, True)
        return 0

    lax.fori_loop(0, (NSCH - 2) // 2, body, 0)
    superchunk(NSCH - 2, 0, True)
    superchunk(NSCH - 1, 1, False)

    pltpu.sync_copy(out_v, out_h.at[pl.ds(wid * B_PER_W, B_PER_W)])


@jax.jit
def _run(ing_flat, user, recipe, ing_table, user_table, user_bias,
         recipe_bias):
    mesh = plsc.VectorSubcoreMesh(core_axis_name="c", subcore_axis_name="s")
    remb_flat, pb_flat = pl.kernel(
        _ing_body,
        out_type=(jax.ShapeDtypeStruct((B * D,), jnp.float32),
                  jax.ShapeDtypeStruct((B,), jnp.float32)),
        mesh=mesh,
        compiler_params=pltpu.CompilerParams(
            needs_layout_passes=False, use_tc_tiling_on_sc=False),
        scratch_types=[
            pltpu.VMEM((CH * L,), jnp.int32),
            pltpu.VMEM((CH * L,), jnp.int32),
            pltpu.VMEM((CH,), jnp.int32),
            pltpu.VMEM((CH,), jnp.int32),
            pltpu.VMEM((CH,), jnp.int32),
            pltpu.VMEM((CH,), jnp.int32),
            pltpu.VMEM((CH * L, D), jnp.float32),
            pltpu.VMEM((CH * L, D), jnp.float32),
            pltpu.VMEM((CH * D,), jnp.float32),
            pltpu.VMEM((CH * D,), jnp.float32),
            pltpu.VMEM((CH,), jnp.float32),
            pltpu.VMEM((CH,), jnp.float32),
            pltpu.VMEM((CH,), jnp.float32),
            pltpu.VMEM((CH,), jnp.float32),
            pltpu.SemaphoreType.DMA,
            pltpu.SemaphoreType.DMA,
            pltpu.SemaphoreType.DMA,
            pltpu.SemaphoreType.DMA,
        ],
    )(ing_flat, user, recipe, ing_table, user_bias, recipe_bias)

    return pl.kernel(
        _user_body,
        out_type=jax.ShapeDtypeStruct((B,), jnp.float32),
        mesh=mesh,
        compiler_params=pltpu.CompilerParams(
            needs_layout_passes=False, use_tc_tiling_on_sc=True),
        scratch_types=[
            pltpu.VMEM((SCH,), jnp.int32),
            pltpu.VMEM((SCH,), jnp.int32),
            pltpu.VMEM((SCH * D,), jnp.float32),
            pltpu.VMEM((SCH * D,), jnp.float32),
            pltpu.VMEM((Q * D, 128), jnp.float32),
            pltpu.VMEM((Q * D, 128), jnp.float32),
            pltpu.VMEM((SCH,), jnp.float32),
            pltpu.VMEM((SCH,), jnp.float32),
            pltpu.VMEM((B_PER_W,), jnp.float32),
            pltpu.VMEM((SCH * 16,), jnp.float32),
            pltpu.SemaphoreType.DMA,
            pltpu.SemaphoreType.DMA,
            pltpu.SemaphoreType.DMA,
            pltpu.SemaphoreType.DMA,
        ],
    )(user_table, user, remb_flat, pb_flat)


def kernel(ingredients, user, recipe, user_table, ing_table, user_bias,
           recipe_bias):
    ing_flat = ingredients.reshape(-1)
    return _run(ing_flat, user, recipe, ing_table, user_table.T,
                user_bias.reshape(-1), recipe_bias.reshape(-1))
